# baseline (speedup 1.0000x reference)
import functools

import jax
import jax.numpy as jnp
from jax import lax
from jax.experimental import pallas as pl
from jax.experimental.pallas import tpu as pltpu
from jax.experimental.pallas import tpu_sc as plsc

N = 10000
E = 320000
D = 128
NC = 2
NS = 16
NW = NC * NS
EPW = E // NW
CH = 80
NCHUNK = EPW // CH
RPT = 624
LANES = 16


CHUNKS = [48] * 13
TAIL = N - NS * RPT
SBUF = 80


def _sc_aggregate_body(u_hbm, src_hbm, dst_hbm, zero_hbm, part_hbm, cnt_hbm,
                       src_v, dst_v, rows2_v, ones_v, zcnt_v, accum, cnt_sh,
                       sem_a, sem_b, sem_sa, sem_sb):
    cid = lax.axis_index("c")
    sid = lax.axis_index("s")
    w = cid * NS + sid
    ones16 = jnp.ones((LANES,), jnp.float32)
    zeros16 = jnp.zeros((LANES,), jnp.float32)
    rows_v = rows2_v.at[0]
    rows_b = rows2_v.at[1]

    for j in range(SBUF // LANES):
        ones_v[pl.ds(j * LANES, LANES)] = ones16
        zcnt_v[pl.ds(j * LANES, LANES)] = zeros16
    pltpu.sync_copy(zero_hbm, rows_v)

    off = 0
    for c in CHUNKS:
        pltpu.sync_copy(rows_v.at[pl.ds(0, c)],
                        accum.at[pl.ds(sid * RPT + off, c)])
        pltpu.sync_copy(zcnt_v.at[pl.ds(0, c)],
                        cnt_sh.at[pl.ds(sid * RPT + off, c)])
        off += c

    @pl.when(sid == NS - 1)
    def _zero_tail():
        pltpu.sync_copy(rows_v.at[pl.ds(0, TAIL)],
                        accum.at[pl.ds(NS * RPT, TAIL)])
        pltpu.sync_copy(zcnt_v.at[pl.ds(0, TAIL)],
                        cnt_sh.at[pl.ds(NS * RPT, TAIL)])

    pltpu.sync_copy(src_hbm.at[pl.ds(w * EPW, EPW)], src_v)
    pltpu.sync_copy(dst_hbm.at[pl.ds(w * EPW, EPW)], dst_v)
    plsc.subcore_barrier()

    ones_ch = ones_v.at[pl.ds(0, CH)]

    def sidx(i):
        return src_v.at[pl.ds(i * CH, CH)]

    def didx(i):
        return dst_v.at[pl.ds(i * CH, CH)]

    pltpu.async_copy(u_hbm.at[sidx(0)], rows_v, sem_a)
    pltpu.async_copy(u_hbm.at[sidx(1)], rows_b, sem_b)

    def pair(g, carry):
        i0 = 2 * g
        i1 = i0 + 1
        pltpu.make_async_copy(u_hbm.at[sidx(i0)], rows_v, sem_a).wait()
        pltpu.async_copy(rows_v, accum.at[didx(i0)], sem_sa, priority=1, add=True)
        pltpu.async_copy(ones_ch, cnt_sh.at[didx(i0)], sem_sa, add=True)
        pltpu.make_async_copy(u_hbm.at[sidx(i1)], rows_b, sem_b).wait()
        pltpu.async_copy(rows_b, accum.at[didx(i1)], sem_sb, priority=1, add=True)
        pltpu.async_copy(ones_ch, cnt_sh.at[didx(i1)], sem_sb, add=True)
        pltpu.make_async_copy(rows_v, accum.at[didx(i0)], sem_sa).wait()
        pltpu.make_async_copy(ones_ch, cnt_sh.at[didx(i0)], sem_sa).wait()

        @pl.when(i0 + 2 < NCHUNK)
        def _next_a():
            pltpu.async_copy(u_hbm.at[sidx(i0 + 2)], rows_v, sem_a)

        pltpu.make_async_copy(rows_b, accum.at[didx(i1)], sem_sb).wait()
        pltpu.make_async_copy(ones_ch, cnt_sh.at[didx(i1)], sem_sb).wait()

        @pl.when(i1 + 2 < NCHUNK)
        def _next_b():
            pltpu.async_copy(u_hbm.at[sidx(i1 + 2)], rows_b, sem_b)

        return carry
    lax.fori_loop(0, NCHUNK // 2, pair, 0)

    last = NCHUNK - 1
    pltpu.make_async_copy(u_hbm.at[sidx(last)], rows_v, sem_a).wait()
    pltpu.sync_copy(rows_v, accum.at[didx(last)], add=True)
    pltpu.sync_copy(ones_ch, cnt_sh.at[didx(last)], add=True)

    plsc.subcore_barrier()

    off = 0
    for c in CHUNKS:
        pltpu.sync_copy(accum.at[pl.ds(sid * RPT + off, c)],
                        rows_v.at[pl.ds(0, c)])
        pltpu.sync_copy(rows_v.at[pl.ds(0, c)],
                        part_hbm.at[cid, pl.ds(sid * RPT + off, c)])
        pltpu.sync_copy(cnt_sh.at[pl.ds(sid * RPT + off, c)],
                        ones_v.at[pl.ds(0, c)])
        pltpu.sync_copy(ones_v.at[pl.ds(0, c)],
                        cnt_hbm.at[pl.ds(cid * N + sid * RPT + off, c)])
        off += c

    @pl.when(sid == NS - 1)
    def _write_tail():
        pltpu.sync_copy(accum.at[pl.ds(NS * RPT, TAIL)],
                        rows_v.at[pl.ds(0, TAIL)])
        pltpu.sync_copy(rows_v.at[pl.ds(0, TAIL)],
                        part_hbm.at[cid, pl.ds(NS * RPT, TAIL)])
        pltpu.sync_copy(cnt_sh.at[pl.ds(NS * RPT, TAIL)],
                        zcnt_v.at[pl.ds(0, TAIL)])
        pltpu.sync_copy(zcnt_v.at[pl.ds(0, TAIL)],
                        cnt_hbm.at[pl.ds(cid * N + NS * RPT, TAIL)])


@jax.jit
def _sc_aggregate(u, src, dst):
    mesh = plsc.VectorSubcoreMesh(core_axis_name="c", subcore_axis_name="s")
    k = pl.kernel(
        _sc_aggregate_body,
        out_type=(
            jax.ShapeDtypeStruct((NC, N, D), jnp.float32),
            jax.ShapeDtypeStruct((NC * N,), jnp.float32),
        ),
        mesh=mesh,
        scratch_types=[
            pltpu.VMEM((EPW,), jnp.int32),
            pltpu.VMEM((EPW,), jnp.int32),
            pltpu.VMEM((2, CH, D), jnp.float32),
            pltpu.VMEM((SBUF,), jnp.float32),
            pltpu.VMEM((SBUF,), jnp.float32),
            pltpu.VMEM_SHARED((N, D), jnp.float32),
            pltpu.VMEM_SHARED((N,), jnp.float32),
            pltpu.SemaphoreType.DMA,
            pltpu.SemaphoreType.DMA,
            pltpu.SemaphoreType.DMA,
            pltpu.SemaphoreType.DMA,
        ],
        compiler_params=pltpu.CompilerParams(needs_layout_passes=False),
    )
    zero = jnp.zeros((CH, D), jnp.float32)
    part, cnt = k(u, src, dst, zero)
    cnt = cnt.reshape(NC, GRID, ROWS_BLK).transpose(1, 0, 2)
    return part, cnt


def _matmul_t(h, w):
    return lax.dot_general(h, w, (((1,), (1,)), ((), ())),
                           preferred_element_type=jnp.float32)


ROWS_BLK = 400
GRID = N // ROWS_BLK


def _pre_body(x_ref, wl_ref, wr_ref, bl_ref, u_ref, z_ref):
    h = x_ref[...]
    u_ref[...] = _matmul_t(h, wl_ref[...])
    z_ref[...] = _matmul_t(h, wr_ref[...]) + bl_ref[...]


@jax.jit
def _tc_pre(x, Wl, Wr, bl):
    return pl.pallas_call(
        _pre_body,
        grid=(GRID,),
        in_specs=[
            pl.BlockSpec((ROWS_BLK, D), lambda i: (i, 0)),
            pl.BlockSpec((D, D), lambda i: (0, 0)),
            pl.BlockSpec((D, D), lambda i: (0, 0)),
            pl.BlockSpec((D,), lambda i: (0,)),
        ],
        out_specs=(
            pl.BlockSpec((ROWS_BLK, D), lambda i: (i, 0)),
            pl.BlockSpec((ROWS_BLK, D), lambda i: (i, 0)),
        ),
        out_shape=(
            jax.ShapeDtypeStruct((N, D), jnp.float32),
            jax.ShapeDtypeStruct((N, D), jnp.float32),
        ),
    )(x, Wl, Wr, bl)


def _mean_combine(part_ref, cnt_ref, z_ref):
    s = part_ref[0] + part_ref[1]
    cnt = jnp.maximum(cnt_ref[0, 0] + cnt_ref[0, 1], 1.0)
    return s / cnt[:, None] + z_ref[...]


def _combine_pre_body(part_ref, cnt_ref, z_ref, wl_ref, wr_ref, bl_ref,
                      h_ref, u_ref, z2_ref):
    h = jnp.maximum(_mean_combine(part_ref, cnt_ref, z_ref), 0.0)
    h_ref[...] = h
    u_ref[...] = _matmul_t(h, wl_ref[...])
    z2_ref[...] = _matmul_t(h, wr_ref[...]) + bl_ref[...]


@jax.jit
def _tc_combine_pre(part, cnt, z, Wl, Wr, bl):
    return pl.pallas_call(
        _combine_pre_body,
        grid=(GRID,),
        in_specs=[
            pl.BlockSpec((NC, ROWS_BLK, D), lambda i: (0, i, 0)),
            pl.BlockSpec((1, NC, ROWS_BLK), lambda i: (i, 0, 0)),
            pl.BlockSpec((ROWS_BLK, D), lambda i: (i, 0)),
            pl.BlockSpec((D, D), lambda i: (0, 0)),
            pl.BlockSpec((D, D), lambda i: (0, 0)),
            pl.BlockSpec((D,), lambda i: (0,)),
        ],
        out_specs=(
            pl.BlockSpec((ROWS_BLK, D), lambda i: (i, 0)),
            pl.BlockSpec((ROWS_BLK, D), lambda i: (i, 0)),
            pl.BlockSpec((ROWS_BLK, D), lambda i: (i, 0)),
        ),
        out_shape=(
            jax.ShapeDtypeStruct((N, D), jnp.float32),
            jax.ShapeDtypeStruct((N, D), jnp.float32),
            jax.ShapeDtypeStruct((N, D), jnp.float32),
        ),
    )(part, cnt, z, Wl, Wr, bl)


def _combine_final_body(part_ref, cnt_ref, z_ref, h_ref):
    h_ref[...] = _mean_combine(part_ref, cnt_ref, z_ref)


@jax.jit
def _tc_combine_final(part, cnt, z):
    return pl.pallas_call(
        _combine_final_body,
        grid=(GRID,),
        in_specs=[
            pl.BlockSpec((NC, ROWS_BLK, D), lambda i: (0, i, 0)),
            pl.BlockSpec((1, NC, ROWS_BLK), lambda i: (i, 0, 0)),
            pl.BlockSpec((ROWS_BLK, D), lambda i: (i, 0)),
        ],
        out_specs=pl.BlockSpec((ROWS_BLK, D), lambda i: (i, 0)),
        out_shape=jax.ShapeDtypeStruct((N, D), jnp.float32),
    )(part, cnt, z)


def kernel(x, edge_index_l0, edge_index_l1, edge_index_l2,
           Wl0, bl0, Wr0, Wl1, bl1, Wr1, Wl2, bl2, Wr2):
    def prep(ei):
        return (ei[0], ei[1])

    s0, d0 = prep(edge_index_l0)
    s1, d1 = prep(edge_index_l1)
    s2, d2 = prep(edge_index_l2)

    u, z = _tc_pre(x, Wl0, Wr0, bl0)
    part, cnt = _sc_aggregate(u, s0, d0)
    h1, u, z = _tc_combine_pre(part, cnt, z, Wl1, Wr1, bl1)
    part, cnt = _sc_aggregate(u, s1, d1)
    h2, u, z = _tc_combine_pre(part, cnt, z, Wl2, Wr2, bl2)
    part, cnt = _sc_aggregate(u, s2, d2)
    h3 = _tc_combine_final(part, cnt, z)
    return (h1, h2, h3)


# speedup vs baseline: 1.0114x; 1.0114x over previous
import functools

import jax
import jax.numpy as jnp
from jax import lax
from jax.experimental import pallas as pl
from jax.experimental.pallas import tpu as pltpu
from jax.experimental.pallas import tpu_sc as plsc

N = 10000
E = 320000
D = 128
NC = 2
NS = 16
NW = NC * NS
EPW = E // NW
CH = 80
NCHUNK = EPW // CH
RPT = 624
LANES = 16


CHUNKS = [48] * 13
TAIL = N - NS * RPT
SBUF = 80


def _sc_aggregate_body(u_hbm, src_hbm, dst_hbm, zero_hbm, part_hbm, cnt_hbm,
                       src_v, dst_v, rows2_v, ones_v, zcnt_v, accum, cnt_sh,
                       sem_a, sem_b, sem_sa, sem_sb):
    cid = lax.axis_index("c")
    sid = lax.axis_index("s")
    w = cid * NS + sid
    ones16 = jnp.ones((LANES,), jnp.float32)
    zeros16 = jnp.zeros((LANES,), jnp.float32)
    rows_v = rows2_v.at[0]
    rows_b = rows2_v.at[1]

    for j in range(SBUF // LANES):
        ones_v[pl.ds(j * LANES, LANES)] = ones16
        zcnt_v[pl.ds(j * LANES, LANES)] = zeros16
    pltpu.sync_copy(zero_hbm, rows_v)

    off = 0
    for c in CHUNKS:
        pltpu.sync_copy(rows_v.at[pl.ds(0, c)],
                        accum.at[pl.ds(sid * RPT + off, c)])
        pltpu.sync_copy(zcnt_v.at[pl.ds(0, c)],
                        cnt_sh.at[pl.ds(sid * RPT + off, c)])
        off += c

    @pl.when(sid == NS - 1)
    def _zero_tail():
        pltpu.sync_copy(rows_v.at[pl.ds(0, TAIL)],
                        accum.at[pl.ds(NS * RPT, TAIL)])
        pltpu.sync_copy(zcnt_v.at[pl.ds(0, TAIL)],
                        cnt_sh.at[pl.ds(NS * RPT, TAIL)])

    pltpu.sync_copy(src_hbm.at[pl.ds(w * EPW, EPW)], src_v)
    pltpu.sync_copy(dst_hbm.at[pl.ds(w * EPW, EPW)], dst_v)
    plsc.subcore_barrier()

    ones_ch = ones_v.at[pl.ds(0, CH)]

    def sidx(i):
        return src_v.at[pl.ds(i * CH, CH)]

    def didx(i):
        return dst_v.at[pl.ds(i * CH, CH)]

    pltpu.async_copy(u_hbm.at[sidx(0)], rows_v, sem_a)
    pltpu.async_copy(u_hbm.at[sidx(1)], rows_b, sem_b)

    def pair(g, carry):
        i0 = 2 * g
        i1 = i0 + 1
        pltpu.make_async_copy(u_hbm.at[sidx(i0)], rows_v, sem_a).wait()
        pltpu.async_copy(rows_v, accum.at[didx(i0)], sem_sa, add=True)
        pltpu.async_copy(ones_ch, cnt_sh.at[didx(i0)], sem_sa, add=True)
        pltpu.make_async_copy(u_hbm.at[sidx(i1)], rows_b, sem_b).wait()
        pltpu.async_copy(rows_b, accum.at[didx(i1)], sem_sb, add=True)
        pltpu.async_copy(ones_ch, cnt_sh.at[didx(i1)], sem_sb, add=True)
        pltpu.make_async_copy(rows_v, accum.at[didx(i0)], sem_sa).wait()
        pltpu.make_async_copy(ones_ch, cnt_sh.at[didx(i0)], sem_sa).wait()

        @pl.when(i0 + 2 < NCHUNK)
        def _next_a():
            pltpu.async_copy(u_hbm.at[sidx(i0 + 2)], rows_v, sem_a)

        pltpu.make_async_copy(rows_b, accum.at[didx(i1)], sem_sb).wait()
        pltpu.make_async_copy(ones_ch, cnt_sh.at[didx(i1)], sem_sb).wait()

        @pl.when(i1 + 2 < NCHUNK)
        def _next_b():
            pltpu.async_copy(u_hbm.at[sidx(i1 + 2)], rows_b, sem_b)

        return carry
    lax.fori_loop(0, NCHUNK // 2, pair, 0)

    last = NCHUNK - 1
    pltpu.make_async_copy(u_hbm.at[sidx(last)], rows_v, sem_a).wait()
    pltpu.sync_copy(rows_v, accum.at[didx(last)], add=True)
    pltpu.sync_copy(ones_ch, cnt_sh.at[didx(last)], add=True)

    plsc.subcore_barrier()

    off = 0
    for c in CHUNKS:
        pltpu.sync_copy(accum.at[pl.ds(sid * RPT + off, c)],
                        rows_v.at[pl.ds(0, c)])
        pltpu.sync_copy(rows_v.at[pl.ds(0, c)],
                        part_hbm.at[cid, pl.ds(sid * RPT + off, c)])
        pltpu.sync_copy(cnt_sh.at[pl.ds(sid * RPT + off, c)],
                        ones_v.at[pl.ds(0, c)])
        pltpu.sync_copy(ones_v.at[pl.ds(0, c)],
                        cnt_hbm.at[pl.ds(cid * N + sid * RPT + off, c)])
        off += c

    @pl.when(sid == NS - 1)
    def _write_tail():
        pltpu.sync_copy(accum.at[pl.ds(NS * RPT, TAIL)],
                        rows_v.at[pl.ds(0, TAIL)])
        pltpu.sync_copy(rows_v.at[pl.ds(0, TAIL)],
                        part_hbm.at[cid, pl.ds(NS * RPT, TAIL)])
        pltpu.sync_copy(cnt_sh.at[pl.ds(NS * RPT, TAIL)],
                        zcnt_v.at[pl.ds(0, TAIL)])
        pltpu.sync_copy(zcnt_v.at[pl.ds(0, TAIL)],
                        cnt_hbm.at[pl.ds(cid * N + NS * RPT, TAIL)])


@jax.jit
def _sc_aggregate(u, src, dst):
    mesh = plsc.VectorSubcoreMesh(core_axis_name="c", subcore_axis_name="s")
    k = pl.kernel(
        _sc_aggregate_body,
        out_type=(
            jax.ShapeDtypeStruct((NC, N, D), jnp.float32),
            jax.ShapeDtypeStruct((NC * N,), jnp.float32),
        ),
        mesh=mesh,
        scratch_types=[
            pltpu.VMEM((EPW,), jnp.int32),
            pltpu.VMEM((EPW,), jnp.int32),
            pltpu.VMEM((2, CH, D), jnp.float32),
            pltpu.VMEM((SBUF,), jnp.float32),
            pltpu.VMEM((SBUF,), jnp.float32),
            pltpu.VMEM_SHARED((N, D), jnp.float32),
            pltpu.VMEM_SHARED((N,), jnp.float32),
            pltpu.SemaphoreType.DMA,
            pltpu.SemaphoreType.DMA,
            pltpu.SemaphoreType.DMA,
            pltpu.SemaphoreType.DMA,
        ],
        compiler_params=pltpu.CompilerParams(needs_layout_passes=False),
    )
    zero = jnp.zeros((CH, D), jnp.float32)
    part, cnt = k(u, src, dst, zero)
    cnt = cnt.reshape(NC, GRID, ROWS_BLK).transpose(1, 0, 2)
    return part, cnt


def _matmul_t(h, w):
    return lax.dot_general(h, w, (((1,), (1,)), ((), ())),
                           preferred_element_type=jnp.float32)


ROWS_BLK = 400
GRID = N // ROWS_BLK


def _u_body(x_ref, wl_ref, u_ref):
    u_ref[...] = _matmul_t(x_ref[...], wl_ref[...])


@jax.jit
def _tc_u(x, Wl):
    return pl.pallas_call(
        _u_body,
        grid=(GRID,),
        in_specs=[
            pl.BlockSpec((ROWS_BLK, D), lambda i: (i, 0)),
            pl.BlockSpec((D, D), lambda i: (0, 0)),
        ],
        out_specs=pl.BlockSpec((ROWS_BLK, D), lambda i: (i, 0)),
        out_shape=jax.ShapeDtypeStruct((N, D), jnp.float32),
    )(x, Wl)


def _z_body(x_ref, wr_ref, bl_ref, z_ref):
    z_ref[...] = _matmul_t(x_ref[...], wr_ref[...]) + bl_ref[...]


@jax.jit
def _tc_z(x, Wr, bl):
    return pl.pallas_call(
        _z_body,
        grid=(GRID,),
        in_specs=[
            pl.BlockSpec((ROWS_BLK, D), lambda i: (i, 0)),
            pl.BlockSpec((D, D), lambda i: (0, 0)),
            pl.BlockSpec((D,), lambda i: (0,)),
        ],
        out_specs=pl.BlockSpec((ROWS_BLK, D), lambda i: (i, 0)),
        out_shape=jax.ShapeDtypeStruct((N, D), jnp.float32),
    )(x, Wr, bl)


def _mean_combine(part_ref, cnt_ref, z_ref):
    s = part_ref[0] + part_ref[1]
    cnt = jnp.maximum(cnt_ref[0, 0] + cnt_ref[0, 1], 1.0)
    return s / cnt[:, None] + z_ref[...]


def _combine_pre_body(part_ref, cnt_ref, z_ref, wl_ref, h_ref, u_ref):
    h = jnp.maximum(_mean_combine(part_ref, cnt_ref, z_ref), 0.0)
    h_ref[...] = h
    u_ref[...] = _matmul_t(h, wl_ref[...])


@jax.jit
def _tc_combine_pre(part, cnt, z, Wl):
    return pl.pallas_call(
        _combine_pre_body,
        grid=(GRID,),
        in_specs=[
            pl.BlockSpec((NC, ROWS_BLK, D), lambda i: (0, i, 0)),
            pl.BlockSpec((1, NC, ROWS_BLK), lambda i: (i, 0, 0)),
            pl.BlockSpec((ROWS_BLK, D), lambda i: (i, 0)),
            pl.BlockSpec((D, D), lambda i: (0, 0)),
        ],
        out_specs=(
            pl.BlockSpec((ROWS_BLK, D), lambda i: (i, 0)),
            pl.BlockSpec((ROWS_BLK, D), lambda i: (i, 0)),
        ),
        out_shape=(
            jax.ShapeDtypeStruct((N, D), jnp.float32),
            jax.ShapeDtypeStruct((N, D), jnp.float32),
        ),
    )(part, cnt, z, Wl)


def _combine_final_body(part_ref, cnt_ref, z_ref, h_ref):
    h_ref[...] = _mean_combine(part_ref, cnt_ref, z_ref)


@jax.jit
def _tc_combine_final(part, cnt, z):
    return pl.pallas_call(
        _combine_final_body,
        grid=(GRID,),
        in_specs=[
            pl.BlockSpec((NC, ROWS_BLK, D), lambda i: (0, i, 0)),
            pl.BlockSpec((1, NC, ROWS_BLK), lambda i: (i, 0, 0)),
            pl.BlockSpec((ROWS_BLK, D), lambda i: (i, 0)),
        ],
        out_specs=pl.BlockSpec((ROWS_BLK, D), lambda i: (i, 0)),
        out_shape=jax.ShapeDtypeStruct((N, D), jnp.float32),
    )(part, cnt, z)


def kernel(x, edge_index_l0, edge_index_l1, edge_index_l2,
           Wl0, bl0, Wr0, Wl1, bl1, Wr1, Wl2, bl2, Wr2):
    def prep(ei):
        return (ei[0], ei[1])

    s0, d0 = prep(edge_index_l0)
    s1, d1 = prep(edge_index_l1)
    s2, d2 = prep(edge_index_l2)

    u = _tc_u(x, Wl0)
    part, cnt = _sc_aggregate(u, s0, d0)
    z = _tc_z(x, Wr0, bl0)
    h1, u = _tc_combine_pre(part, cnt, z, Wl1)
    part, cnt = _sc_aggregate(u, s1, d1)
    z = _tc_z(h1, Wr1, bl1)
    h2, u = _tc_combine_pre(part, cnt, z, Wl2)
    part, cnt = _sc_aggregate(u, s2, d2)
    z = _tc_z(h2, Wr2, bl2)
    h3 = _tc_combine_final(part, cnt, z)
    return (h1, h2, h3)
